# baseline (speedup 1.0000x reference)
import functools

import jax
import jax.numpy as jnp
from jax import lax
from jax.experimental import pallas as pl
from jax.experimental.pallas import tpu as pltpu
from jax.experimental.pallas import tpu_sc as plsc


def _make_argmax_body(row_stride, cols, sub):
    nsteps = cols // sub

    def body(x_ref, o_ref):
        m0 = x_ref[:, 0:sub, :]
        i0 = lax.broadcasted_iota(jnp.int32, m0.shape, 1)

        def step(k, carry):
            m, i = carry
            chunk = x_ref[:, pl.ds(k * sub, sub), :]
            upd = chunk > m
            m = jnp.where(upd, chunk, m)
            i = jnp.where(upd, i0 + k * sub, i)
            return m, i

        m, i = lax.fori_loop(1, nsteps, step, (m0, i0), unroll=2)
        mf = jnp.max(m, axis=1, keepdims=True)
        am = jnp.min(jnp.where(m == mf, i, jnp.int32(2**30)), axis=1)
        o_ref[0, 0, :] = am[0, :] * row_stride + am[1, :]

    return body


def _tc_argmax(st, batch_tc, block_cols, row_stride):
    two, cols, _ = st.shape
    grid = batch_tc // block_cols
    out = pl.pallas_call(
        _make_argmax_body(row_stride, cols, 8),
        grid=(grid,),
        in_specs=[pl.BlockSpec((two, cols, block_cols), lambda i: (0, 0, i))],
        out_specs=pl.BlockSpec((1, 1, block_cols), lambda i: (i, 0, 0)),
        out_shape=jax.ShapeDtypeStruct((grid, 1, block_cols), jnp.int32),
    )(st)
    return out.reshape(batch_tc)


_CHUNK = 128
_LANE_TILE = 128
_SUB_TILE = 8
_NET = 25


def _make_sc_argmax_gather(e, acts, etiles, sc_tile_base, tpw):
    info = plsc.get_sparse_core_info()
    nc, ns, L = info.num_cores, info.num_subcores, info.num_lanes
    nw = nc * ns
    nchunks = etiles // _NET
    ngrp = _LANE_TILE // L
    sc_batch = nw * tpw * _LANE_TILE
    mesh = plsc.VectorSubcoreMesh(core_axis_name="c", subcore_axis_name="s")

    @functools.partial(
        pl.kernel,
        mesh=mesh,
        out_type=jax.ShapeDtypeStruct((sc_batch,), jnp.float32),
        scratch_types=[
            pltpu.VMEM((_NET, _SUB_TILE, _LANE_TILE), jnp.float32),
            pltpu.VMEM((_NET, _SUB_TILE, _LANE_TILE), jnp.float32),
            pltpu.VMEM((_LANE_TILE,), jnp.int32),
            pltpu.VMEM((_LANE_TILE,), jnp.int32),
            pltpu.VMEM((_LANE_TILE,), jnp.float32),
            pltpu.SemaphoreType.DMA,
            pltpu.SemaphoreType.DMA,
        ],
        compiler_params=pltpu.CompilerParams(use_tc_tiling_on_sc=False),
    )
    def sc_argmax(n5_hbm, a_hbm, table_hbm, out_hbm,
                  buf0, buf1, a_v, fl_v, out_v, sem0, sem1):
        wid = lax.axis_index("s") * nc + lax.axis_index("c")
        bufs = (buf0, buf1)
        sems = (sem0, sem1)
        minf = jnp.full((L,), -jnp.inf, jnp.float32)
        izero = jnp.full((L,), 0, jnp.int32)
        ve8 = [jnp.full((L,), j, jnp.int32) for j in range(_SUB_TILE)]

        for t in range(tpw):
            bt = sc_tile_base + wid * tpw + t
            pltpu.sync_copy(a_hbm.at[pl.ds(bt * _LANE_TILE, _LANE_TILE)], a_v)
            am = [None, None]
            for c in range(2):
                cp = [None] * nchunks
                cp[0] = pltpu.async_copy(
                    n5_hbm.at[c, pl.ds(0, _NET), bt, :, :], bufs[0], sems[0])
                m = [minf] * ngrp
                i = [izero] * ngrp
                for k in range(nchunks):
                    if k + 1 < nchunks:
                        cp[k + 1] = pltpu.async_copy(
                            n5_hbm.at[c, pl.ds((k + 1) * _NET, _NET), bt, :, :],
                            bufs[(k + 1) % 2], sems[(k + 1) % 2])
                    cp[k].wait()
                    buf = bufs[k % 2]

                    def step(et, carry, _k=k, _buf=buf):
                        cm, ci = list(carry[0]), list(carry[1])
                        base = jnp.full((L,), (_k * _NET + et) * _SUB_TILE,
                                        jnp.int32)
                        for e8 in range(_SUB_TILE):
                            cand = base + ve8[e8]
                            for g in range(ngrp):
                                v = _buf[et, e8, pl.ds(g * L, L)]
                                upd = v > cm[g]
                                cm[g] = jnp.where(upd, v, cm[g])
                                ci[g] = jnp.where(upd, cand, ci[g])
                        return tuple(cm), tuple(ci)

                    m, i = lax.fori_loop(0, _NET, step, (tuple(m), tuple(i)))
                    m, i = list(m), list(i)
                am[c] = i
            for g in range(ngrp):
                sl = pl.ds(g * L, L)
                fl_v[sl] = am[0][g] * (e * acts) + a_v[sl] * e + am[1][g]
            pltpu.async_copy(table_hbm.at[fl_v], out_v, sem0).wait()
            pltpu.sync_copy(
                out_v,
                out_hbm.at[pl.ds((wid * tpw + t) * _LANE_TILE, _LANE_TILE)])

    return sc_argmax


def _make_sc_gather(batch, e, acts):
    info = plsc.get_sparse_core_info()
    nc, ns, L = info.num_cores, info.num_subcores, info.num_lanes
    nw = nc * ns
    bpw = batch // nw
    nchunk = bpw // _CHUNK
    mesh = plsc.VectorSubcoreMesh(core_axis_name="c", subcore_axis_name="s")

    @functools.partial(
        pl.kernel,
        mesh=mesh,
        out_type=jax.ShapeDtypeStruct((batch,), jnp.float32),
        scratch_types=[
            pltpu.VMEM((nchunk, _CHUNK), jnp.int32),
            pltpu.VMEM((nchunk, _CHUNK), jnp.int32),
            pltpu.VMEM((nchunk, _CHUNK), jnp.float32),
            pltpu.SemaphoreType.DMA,
        ],
        compiler_params=pltpu.CompilerParams(use_tc_tiling_on_sc=False),
    )
    def sc_gather(comb_hbm, a_hbm, table_hbm, out_hbm, idx_v, a_v, out_v, sem):
        wid = lax.axis_index("s") * nc + lax.axis_index("c")
        base = wid * bpw
        for c in range(nchunk):
            pltpu.sync_copy(comb_hbm.at[pl.ds(base + c * _CHUNK, _CHUNK)], idx_v.at[c])
            pltpu.sync_copy(a_hbm.at[pl.ds(base + c * _CHUNK, _CHUNK)], a_v.at[c])
        for c in range(nchunk):
            for o in range(_CHUNK // L):
                sl = pl.ds(o * L, L)
                idx_v[c, sl] = idx_v[c, sl] + a_v[c, sl] * e
        copies = [
            pltpu.async_copy(table_hbm.at[idx_v.at[c]], out_v.at[c], sem)
            for c in range(nchunk)
        ]
        for cp in copies:
            cp.wait()
        for c in range(nchunk):
            pltpu.sync_copy(out_v.at[c], out_hbm.at[pl.ds(base + c * _CHUNK, _CHUNK)])

    return sc_gather


_TPW = 1


def kernel(s, a, env_size, table):
    batch = s.shape[0]
    e = s.shape[2]
    acts = table.shape[2]
    etiles = e // _SUB_TILE
    sc_batch = 32 * _TPW * _LANE_TILE
    batch_tc = batch - sc_batch
    sc_tile_base = batch_tc // _LANE_TILE

    st = s.transpose(1, 2, 0)
    n5 = st.reshape(2, etiles, _SUB_TILE, batch // _LANE_TILE, _LANE_TILE)
    n5 = n5.transpose(0, 1, 3, 2, 4)
    a32 = a.astype(jnp.int32)
    tflat = table.transpose(0, 2, 1).reshape(-1)

    comb_tc = _tc_argmax(st, batch_tc, 1024, e * acts)
    sc_argmax = _make_sc_argmax_gather(e, acts, etiles, sc_tile_base, _TPW)
    out_sc = sc_argmax(n5, a32, tflat)
    sc_gather = _make_sc_gather(batch_tc, e, acts)
    out_tc = sc_gather(comb_tc, a32, tflat)
    return jnp.concatenate([out_tc, out_sc])


# speedup vs baseline: 1.4544x; 1.4544x over previous
import functools

import jax
import jax.numpy as jnp
from jax import lax
from jax.experimental import pallas as pl
from jax.experimental.pallas import tpu as pltpu
from jax.experimental.pallas import tpu_sc as plsc


def _make_argmax_body(row_stride, cols, sub):
    nsteps = cols // sub

    def body(x_ref, o_ref):
        m0 = x_ref[:, 0:sub, :]
        i0 = lax.broadcasted_iota(jnp.int32, m0.shape, 1)

        def step(k, carry):
            m, i = carry
            chunk = x_ref[:, pl.ds(k * sub, sub), :]
            upd = chunk > m
            m = jnp.where(upd, chunk, m)
            i = jnp.where(upd, i0 + k * sub, i)
            return m, i

        m, i = lax.fori_loop(1, nsteps, step, (m0, i0), unroll=2)
        mf = jnp.max(m, axis=1, keepdims=True)
        am = jnp.min(jnp.where(m == mf, i, jnp.int32(2**30)), axis=1)
        o_ref[0, 0, :] = am[0, :] * row_stride + am[1, :]

    return body


def _tc_argmax(st, block_cols, row_stride):
    two, cols, batch = st.shape
    grid = batch // block_cols
    out = pl.pallas_call(
        _make_argmax_body(row_stride, cols, 8),
        grid=(grid,),
        in_specs=[pl.BlockSpec((two, cols, block_cols), lambda i: (0, 0, i))],
        out_specs=pl.BlockSpec((1, 1, block_cols), lambda i: (i, 0, 0)),
        out_shape=jax.ShapeDtypeStruct((grid, 1, block_cols), jnp.int32),
    )(st)
    return out.reshape(batch)


_CHUNK = 128


def _make_sc_gather(batch, act_stride):
    info = plsc.get_sparse_core_info()
    nc, ns, L = info.num_cores, info.num_subcores, info.num_lanes
    nw = nc * ns
    bpw = batch // nw
    nchunk = bpw // _CHUNK
    mesh = plsc.VectorSubcoreMesh(core_axis_name="c", subcore_axis_name="s")

    @functools.partial(
        pl.kernel,
        mesh=mesh,
        out_type=jax.ShapeDtypeStruct((batch,), jnp.float32),
        scratch_types=[
            pltpu.VMEM((nchunk, _CHUNK), jnp.int32),
            pltpu.VMEM((nchunk, _CHUNK), jnp.int32),
            pltpu.VMEM((nchunk, _CHUNK), jnp.float32),
            pltpu.SemaphoreType.DMA,
        ],
        compiler_params=pltpu.CompilerParams(use_tc_tiling_on_sc=False),
    )
    def sc_gather(comb_hbm, a_hbm, table_hbm, out_hbm, idx_v, a_v, out_v, sem):
        wid = lax.axis_index("s") * nc + lax.axis_index("c")
        base = wid * bpw
        for c in range(nchunk):
            pltpu.sync_copy(comb_hbm.at[pl.ds(base + c * _CHUNK, _CHUNK)], idx_v.at[c])
            pltpu.sync_copy(a_hbm.at[pl.ds(base + c * _CHUNK, _CHUNK)], a_v.at[c])
        for c in range(nchunk):
            for o in range(_CHUNK // L):
                sl = pl.ds(o * L, L)
                idx_v[c, sl] = idx_v[c, sl] + a_v[c, sl] * act_stride
        copies = [
            pltpu.async_copy(table_hbm.at[idx_v.at[c]], out_v.at[c], sem)
            for c in range(nchunk)
        ]
        for cp in copies:
            cp.wait()
        for c in range(nchunk):
            pltpu.sync_copy(out_v.at[c], out_hbm.at[pl.ds(base + c * _CHUNK, _CHUNK)])

    return sc_gather


def kernel(s, a, env_size, table):
    batch = s.shape[0]
    e = s.shape[2]
    acts = table.shape[2]
    st = s.transpose(1, 2, 0)
    comb = _tc_argmax(st, 1024, e * acts)
    a32 = a.astype(jnp.int32)
    tflat = table.transpose(0, 2, 1).reshape(-1)
    sc_gather = _make_sc_gather(batch, e)
    return sc_gather(comb, a32, tflat)


# speedup vs baseline: 1.4769x; 1.0155x over previous
import functools

import jax
import jax.numpy as jnp
from jax import lax
from jax.experimental import pallas as pl
from jax.experimental.pallas import tpu as pltpu
from jax.experimental.pallas import tpu_sc as plsc


def _make_argmax_body(row_stride, cols, sub):
    del cols, sub

    def body(x_ref, o_ref):
        v = x_ref[...]
        m = jnp.max(v, axis=1, keepdims=True)
        col = lax.broadcasted_iota(jnp.int32, v.shape, 1)
        am = jnp.min(jnp.where(v == m, col, jnp.int32(2**30)), axis=1)
        o_ref[0, 0, :] = am[0, :] * row_stride + am[1, :]

    return body


def _tc_argmax(st, block_cols, row_stride):
    two, cols, batch = st.shape
    grid = batch // block_cols
    out = pl.pallas_call(
        _make_argmax_body(row_stride, cols, 8),
        grid=(grid,),
        in_specs=[pl.BlockSpec((two, cols, block_cols), lambda i: (0, 0, i))],
        out_specs=pl.BlockSpec((1, 1, block_cols), lambda i: (i, 0, 0)),
        out_shape=jax.ShapeDtypeStruct((grid, 1, block_cols), jnp.int32),
    )(st)
    return out.reshape(batch)


_CHUNK = 128


def _make_sc_gather(batch, act_stride):
    info = plsc.get_sparse_core_info()
    nc, ns, L = info.num_cores, info.num_subcores, info.num_lanes
    nw = nc * ns
    bpw = batch // nw
    nchunk = bpw // _CHUNK
    mesh = plsc.VectorSubcoreMesh(core_axis_name="c", subcore_axis_name="s")

    @functools.partial(
        pl.kernel,
        mesh=mesh,
        out_type=jax.ShapeDtypeStruct((batch,), jnp.float32),
        scratch_types=[
            pltpu.VMEM((nchunk, _CHUNK), jnp.int32),
            pltpu.VMEM((nchunk, _CHUNK), jnp.int32),
            pltpu.VMEM((nchunk, _CHUNK), jnp.float32),
            pltpu.SemaphoreType.DMA,
        ],
        compiler_params=pltpu.CompilerParams(use_tc_tiling_on_sc=False),
    )
    def sc_gather(comb_hbm, a_hbm, table_hbm, out_hbm, idx_v, a_v, out_v, sem):
        wid = lax.axis_index("s") * nc + lax.axis_index("c")
        base = wid * bpw
        for c in range(nchunk):
            pltpu.sync_copy(comb_hbm.at[pl.ds(base + c * _CHUNK, _CHUNK)], idx_v.at[c])
            pltpu.sync_copy(a_hbm.at[pl.ds(base + c * _CHUNK, _CHUNK)], a_v.at[c])
        for c in range(nchunk):
            for o in range(_CHUNK // L):
                sl = pl.ds(o * L, L)
                idx_v[c, sl] = idx_v[c, sl] + a_v[c, sl] * act_stride
        copies = [
            pltpu.async_copy(table_hbm.at[idx_v.at[c]], out_v.at[c], sem)
            for c in range(nchunk)
        ]
        for cp in copies:
            cp.wait()
        for c in range(nchunk):
            pltpu.sync_copy(out_v.at[c], out_hbm.at[pl.ds(base + c * _CHUNK, _CHUNK)])

    return sc_gather


def kernel(s, a, env_size, table):
    batch = s.shape[0]
    e = s.shape[2]
    acts = table.shape[2]
    st = s.transpose(1, 2, 0)
    comb = _tc_argmax(st, 1024, e * acts)
    a32 = a.astype(jnp.int32)
    tflat = table.transpose(0, 2, 1).reshape(-1)
    sc_gather = _make_sc_gather(batch, e)
    return sc_gather(comb, a32, tflat)
